# Initial kernel scaffold; baseline (speedup 1.0000x reference)
import jax
import jax.numpy as jnp
from jax.experimental import pallas as pl


def kernel(x, n_id0, res_n_id0, edge_index0, res_n_id1, edge_index1, W1, att_src1, att_dst1, b1, W3, att_src3, att_dst3, b3):
    raise NotImplementedError("write your pallas kernel here")


# speedup vs baseline: 3.3422x; 3.3422x over previous
import functools

import jax
import jax.numpy as jnp
from jax.experimental import pallas as pl
from jax.experimental.pallas import tpu as pltpu

H1, C1 = 12, 128
N0 = 10000
M0, E0 = 5000, 320000
M1, E1 = 1000, 64000
D_IN = 128
KMAX = 2048


def _mm_attn_kernel(xs_ref, xd_ref, w_ref, asrc_att_ref, adst_att_ref,
                    hs_ref, asrc_ref, adst_ref):
    w = w_ref[...]
    hs = jnp.dot(xs_ref[...], w, preferred_element_type=jnp.float32)
    hs_ref[...] = hs
    bm = hs.shape[0]
    a_s = (hs.reshape(bm, H1, C1) * asrc_att_ref[...][None]).sum(-1)
    asrc_ref[...] = jnp.concatenate(
        [a_s, jnp.zeros((bm, 16 - H1), jnp.float32)], axis=-1)
    hd = jnp.dot(xd_ref[...], w, preferred_element_type=jnp.float32)
    a_d = (hd.reshape(bm, H1, C1) * adst_att_ref[...][None]).sum(-1)
    adst_ref[...] = jnp.concatenate(
        [a_d, jnp.zeros((bm, 16 - H1), jnp.float32)], axis=-1)


def _stage1(xs, xd, W1, att_src1, att_dst1):
    bm = 1000
    grid = (M0 // bm,)
    return pl.pallas_call(
        _mm_attn_kernel,
        grid=grid,
        in_specs=[
            pl.BlockSpec((bm, D_IN), lambda i: (i, 0)),
            pl.BlockSpec((bm, D_IN), lambda i: (i, 0)),
            pl.BlockSpec((D_IN, H1 * C1), lambda i: (0, 0)),
            pl.BlockSpec((H1, C1), lambda i: (0, 0)),
            pl.BlockSpec((H1, C1), lambda i: (0, 0)),
        ],
        out_specs=[
            pl.BlockSpec((bm, H1 * C1), lambda i: (i, 0)),
            pl.BlockSpec((bm, 16), lambda i: (i, 0)),
            pl.BlockSpec((bm, 16), lambda i: (i, 0)),
        ],
        out_shape=[
            jax.ShapeDtypeStruct((M0, H1 * C1), jnp.float32),
            jax.ShapeDtypeStruct((M0, 16), jnp.float32),
            jax.ShapeDtypeStruct((M0, 16), jnp.float32),
        ],
    )(xs, xd, W1, att_src1, att_dst1)


def kernel(x, n_id0, res_n_id0, edge_index0, res_n_id1, edge_index1,
           W1, att_src1, att_dst1, b1, W3, att_src3, att_dst3, b3):
    x2 = x[0]

    idx_s = n_id0[:M0]
    idx_d = n_id0[res_n_id0]
    needed = jnp.zeros((M0,), jnp.bool_).at[res_n_id1].set(True)
    needed = needed | (jnp.arange(M0, dtype=jnp.int32) < M1)
    cum = jnp.cumsum(needed.astype(jnp.int32))
    remap = jnp.where(needed, cum - 1, KMAX)
    src0, dst0 = edge_index0[0], edge_index0[1]
    cid = remap[dst0]

    xs = x2[idx_s]
    xd = x2[idx_d]

    hs, asrc, adst = _stage1(xs, xd, W1, att_src1, att_dst1)
    asrc = asrc[:, :H1]
    adst = adst[:, :H1]

    al = asrc[src0] + adst[dst0]
    al = jnp.where(al >= 0, al, 0.2 * al)
    ex = jnp.exp(al)
    denom = jax.ops.segment_sum(ex, cid, num_segments=KMAX + 1)
    coef = ex / (denom[cid] + 1e-16)

    msg = hs[src0].reshape(E0, H1, C1) * coef[..., None]
    out0c = jax.ops.segment_sum(msg.reshape(E0, H1 * C1), cid,
                                num_segments=KMAX + 1)[:KMAX]

    h2 = jnp.maximum(out0c + b1[None, :], 0.0) @ W3
    h2v = h2[:, 0]

    hsrc2 = h2v[:M1]
    hdst2 = h2v[remap[res_n_id1]]
    as3 = hsrc2 * att_src3[0, 0]
    ad3 = hdst2 * att_dst3[0, 0]
    src1, dst1 = edge_index1[0], edge_index1[1]
    al2 = as3[src1] + ad3[dst1]
    al2 = jnp.where(al2 >= 0, al2, 0.2 * al2)
    ex2 = jnp.exp(al2)
    den2 = jax.ops.segment_sum(ex2, dst1, num_segments=M1)
    num2 = jax.ops.segment_sum(ex2 * hsrc2[src1], dst1, num_segments=M1)
    out = num2 / (den2 + 1e-16) + b3[0]
    return out.reshape(1, M1, 1)
